# baseline (speedup 1.0000x reference)
import functools

import jax
import jax.numpy as jnp
from jax import lax
from jax.experimental import pallas as pl
from jax.experimental.pallas import tpu as pltpu
from jax.experimental.pallas import tpu_sc as plsc

_L = 16
_NC = 2
_NS = 16


def _row_gather(table2, keypoints, consts, B, C, N, H, W):
    NW = _NC * _NS
    HW = H * W
    PW = (B * N) // NW
    w_per_b = NW // B
    n_grp = PW // _L
    CH = 16
    n_chunk = PW // CH
    mesh = plsc.VectorSubcoreMesh(
        core_axis_name="c", subcore_axis_name="s",
        num_cores=_NC, num_subcores=_NS)

    @functools.partial(
        pl.kernel,
        out_type=jax.ShapeDtypeStruct((B * N * 2, 128), jnp.float32),
        mesh=mesh,
        compiler_params=pltpu.CompilerParams(needs_layout_passes=False),
        scratch_types=[
            pltpu.VMEM((PW, 3), jnp.float32),
            pltpu.VMEM((2, _L), jnp.float32),
            pltpu.VMEM((PW * 8,), jnp.int32),
            pltpu.VMEM((PW * 4,), jnp.float32),
            pltpu.VMEM((CH * 8, 128), jnp.float32),
            pltpu.VMEM((CH * 8, 128), jnp.float32),
            pltpu.VMEM((CH * 2, 128), jnp.float32),
            pltpu.VMEM((CH * 2, 128), jnp.float32),
            pltpu.SemaphoreType.DMA,
            pltpu.SemaphoreType.DMA,
            pltpu.SemaphoreType.DMA,
            pltpu.SemaphoreType.DMA,
        ],
    )
    def k(tab_hbm, kp_hbm, consts_hbm, out_hbm, kp_v, consts_v, ridx, wgt_v,
          rows0, rows1, out0, out1, sr0, sr1, so0, so1):
        wid = lax.axis_index("s") * _NC + lax.axis_index("c")
        b = wid // w_per_b
        p0 = (wid % w_per_b) * PW

        pltpu.sync_copy(kp_hbm.at[b, pl.ds(p0, PW)], kp_v)
        pltpu.sync_copy(consts_hbm, consts_v)
        stride_v = consts_v[0, :]
        scale_v = consts_v[1, :]
        lane = lax.iota(jnp.int32, _L)
        zcol = jnp.zeros((_L,), jnp.int32)
        base_row = 2 * b * HW

        def prep_body(g, carry):
            pos = lane + g * _L
            xs = plsc.load_gather(kp_v, [pos, zcol])
            ys = plsc.load_gather(kp_v, [pos, zcol + 1])
            x = (xs - 0.0) / jnp.float32(0.05) / stride_v
            y = (ys - jnp.float32(-40.0)) / jnp.float32(0.05) / stride_v
            x0t = x.astype(jnp.int32)
            y0t = y.astype(jnp.int32)
            x0c = jnp.clip(x0t, 0, W - 1)
            x1c = jnp.clip(x0t + 1, 0, W - 1)
            y0c = jnp.clip(y0t, 0, H - 1)
            y1c = jnp.clip(y0t + 1, 0, H - 1)
            x0f = x0c.astype(jnp.float32)
            x1f = x1c.astype(jnp.float32)
            y0f = y0c.astype(jnp.float32)
            y1f = y1c.astype(jnp.float32)
            gx = x1f - x
            fx = x - x0f
            gy = y1f - y
            fy = y - y0f
            ra = base_row + 2 * (y0c * W + x0c)
            rb = base_row + 2 * (y1c * W + x0c)
            rc = base_row + 2 * (y0c * W + x1c)
            rd = base_row + 2 * (y1c * W + x1c)
            pos8 = lane * 8 + g * (8 * _L)
            plsc.store_scatter(ridx, [pos8 + 0], ra)
            plsc.store_scatter(ridx, [pos8 + 1], ra + 1)
            plsc.store_scatter(ridx, [pos8 + 2], rb)
            plsc.store_scatter(ridx, [pos8 + 3], rb + 1)
            plsc.store_scatter(ridx, [pos8 + 4], rc)
            plsc.store_scatter(ridx, [pos8 + 5], rc + 1)
            plsc.store_scatter(ridx, [pos8 + 6], rd)
            plsc.store_scatter(ridx, [pos8 + 7], rd + 1)
            pos4 = lane * 4 + g * (4 * _L)
            plsc.store_scatter(wgt_v, [pos4 + 0], gx * gy * scale_v)
            plsc.store_scatter(wgt_v, [pos4 + 1], gx * fy * scale_v)
            plsc.store_scatter(wgt_v, [pos4 + 2], fx * gy * scale_v)
            plsc.store_scatter(wgt_v, [pos4 + 3], fx * fy * scale_v)
            return carry

        lax.fori_loop(0, n_grp, prep_body, 0)

        pltpu.async_copy(tab_hbm.at[ridx.at[pl.ds(0, CH * 8)]], rows0, sr0)
        pltpu.async_copy(
            tab_hbm.at[ridx.at[pl.ds(CH * 8, CH * 8)]], rows1, sr1)

        bufs = ((rows0, out0, sr0, so0), (rows1, out1, sr1, so1))
        zvec = jnp.zeros((_L,), jnp.int32)

        def chunk_body(sidx, carry):
            for u in range(2):
                rows, outb, sr, so = bufs[u]
                ch = sidx * 2 + u
                pltpu.make_async_copy(
                    tab_hbm.at[ridx.at[pl.ds(0, CH * 8)]], rows, sr).wait()

                @pl.when(sidx >= 1)
                def _wait_prev_store():
                    pltpu.make_async_copy(
                        outb, out_hbm.at[pl.ds(0, CH * 2)], so).wait()

                wbase = ch * (CH * 4)

                def pt_body(pp, carry3):
                    wq = zvec + (wbase + 4 * pp)
                    w0 = plsc.load_gather(wgt_v, [wq])
                    w1 = plsc.load_gather(wgt_v, [wq + 1])
                    w2 = plsc.load_gather(wgt_v, [wq + 2])
                    w3 = plsc.load_gather(wgt_v, [wq + 3])
                    r8 = 8 * pp
                    for hh in range(2):
                        for rr in range(8):
                            sl = pl.ds(rr * _L, _L)
                            acc = rows[r8 + hh, sl] * w0
                            acc = acc + rows[r8 + 2 + hh, sl] * w1
                            acc = acc + rows[r8 + 4 + hh, sl] * w2
                            acc = acc + rows[r8 + 6 + hh, sl] * w3
                            outb[2 * pp + hh, sl] = acc
                    return carry3

                lax.fori_loop(0, CH, pt_body, 0)

                @pl.when(ch + 2 < n_chunk)
                def _prefetch_next():
                    pltpu.async_copy(
                        tab_hbm.at[ridx.at[pl.ds((ch + 2) * (CH * 8), CH * 8)]],
                        rows, sr)

                rowbase = (b * N + p0 + ch * CH) * 2
                pltpu.async_copy(outb, out_hbm.at[pl.ds(rowbase, CH * 2)], so)
            return carry

        lax.fori_loop(0, n_chunk // 2, chunk_body, 0)
        pltpu.make_async_copy(out0, out_hbm.at[pl.ds(0, CH * 2)], so0).wait()
        pltpu.make_async_copy(out1, out_hbm.at[pl.ds(0, CH * 2)], so1).wait()

    return k(table2, keypoints, consts)


def kernel(keypoints, bev_features, batch_size, bev_stride):
    B, N, _ = keypoints.shape
    _, C, H, W = bev_features.shape
    stride_f = jnp.asarray(bev_stride, jnp.float32)
    scale_f = jnp.asarray(batch_size, jnp.float32) / B
    consts = jnp.stack([jnp.full((_L,), 1.0, jnp.float32) * stride_f,
                        jnp.full((_L,), 1.0, jnp.float32) * scale_f])
    table2 = (bev_features
              .transpose(0, 2, 3, 1)
              .reshape(B * H * W * 2, 128))
    out = _row_gather(table2, keypoints, consts, B, C, N, H, W)
    return out.reshape(B, N, C)


# speedup vs baseline: 1.0488x; 1.0488x over previous
import functools

import jax
import jax.numpy as jnp
from jax import lax
from jax.experimental import pallas as pl
from jax.experimental.pallas import tpu as pltpu
from jax.experimental.pallas import tpu_sc as plsc

_L = 16
_NC = 2
_NS = 16


def _row_gather(table2, keypoints, consts, B, C, N, H, W):
    NW = _NC * _NS
    HW = H * W
    PW = (B * N) // NW
    w_per_b = NW // B
    n_grp = PW // _L
    CH = 16
    n_chunk = PW // CH
    mesh = plsc.VectorSubcoreMesh(
        core_axis_name="c", subcore_axis_name="s",
        num_cores=_NC, num_subcores=_NS)

    @functools.partial(
        pl.kernel,
        out_type=jax.ShapeDtypeStruct((B * N * 2, 128), jnp.float32),
        mesh=mesh,
        compiler_params=pltpu.CompilerParams(needs_layout_passes=False),
        scratch_types=[
            pltpu.VMEM((PW, 3), jnp.float32),
            pltpu.VMEM((2, _L), jnp.float32),
            pltpu.VMEM((PW * 8,), jnp.int32),
            pltpu.VMEM((PW * 4,), jnp.float32),
            pltpu.VMEM((CH * 8, 128), jnp.float32),
            pltpu.VMEM((CH * 8, 128), jnp.float32),
            pltpu.VMEM((CH * 2, 128), jnp.float32),
            pltpu.VMEM((CH * 2, 128), jnp.float32),
            pltpu.SemaphoreType.DMA,
            pltpu.SemaphoreType.DMA,
            pltpu.SemaphoreType.DMA,
            pltpu.SemaphoreType.DMA,
        ],
    )
    def k(tab_hbm, kp_hbm, consts_hbm, out_hbm, kp_v, consts_v, ridx, wgt_v,
          rows0, rows1, out0, out1, sr0, sr1, so0, so1):
        wid = lax.axis_index("s") * _NC + lax.axis_index("c")
        b = wid // w_per_b
        p0 = (wid % w_per_b) * PW

        pltpu.sync_copy(kp_hbm.at[b, pl.ds(p0, PW)], kp_v)
        pltpu.sync_copy(consts_hbm, consts_v)
        stride_v = consts_v[0, :]
        scale_v = consts_v[1, :]
        lane = lax.iota(jnp.int32, _L)
        zcol = jnp.zeros((_L,), jnp.int32)
        base_row = 2 * b * HW

        def prep_body(g, carry):
            pos = lane + g * _L
            xs = plsc.load_gather(kp_v, [pos, zcol])
            ys = plsc.load_gather(kp_v, [pos, zcol + 1])
            x = (xs - 0.0) / jnp.float32(0.05) / stride_v
            y = (ys - jnp.float32(-40.0)) / jnp.float32(0.05) / stride_v
            x0t = x.astype(jnp.int32)
            y0t = y.astype(jnp.int32)
            x0c = jnp.clip(x0t, 0, W - 1)
            x1c = jnp.clip(x0t + 1, 0, W - 1)
            y0c = jnp.clip(y0t, 0, H - 1)
            y1c = jnp.clip(y0t + 1, 0, H - 1)
            x0f = x0c.astype(jnp.float32)
            x1f = x1c.astype(jnp.float32)
            y0f = y0c.astype(jnp.float32)
            y1f = y1c.astype(jnp.float32)
            gx = x1f - x
            fx = x - x0f
            gy = y1f - y
            fy = y - y0f
            ra = base_row + 2 * (y0c * W + x0c)
            rb = base_row + 2 * (y1c * W + x0c)
            rc = base_row + 2 * (y0c * W + x1c)
            rd = base_row + 2 * (y1c * W + x1c)
            pos8 = lane * 8 + g * (8 * _L)
            plsc.store_scatter(ridx, [pos8 + 0], ra)
            plsc.store_scatter(ridx, [pos8 + 1], ra + 1)
            plsc.store_scatter(ridx, [pos8 + 2], rb)
            plsc.store_scatter(ridx, [pos8 + 3], rb + 1)
            plsc.store_scatter(ridx, [pos8 + 4], rc)
            plsc.store_scatter(ridx, [pos8 + 5], rc + 1)
            plsc.store_scatter(ridx, [pos8 + 6], rd)
            plsc.store_scatter(ridx, [pos8 + 7], rd + 1)
            pos4 = lane * 4 + g * (4 * _L)
            plsc.store_scatter(wgt_v, [pos4 + 0], gx * gy * scale_v)
            plsc.store_scatter(wgt_v, [pos4 + 1], gx * fy * scale_v)
            plsc.store_scatter(wgt_v, [pos4 + 2], fx * gy * scale_v)
            plsc.store_scatter(wgt_v, [pos4 + 3], fx * fy * scale_v)
            return carry

        lax.fori_loop(0, n_grp, prep_body, 0)

        pltpu.async_copy(tab_hbm.at[ridx.at[pl.ds(0, CH * 8)]], rows0, sr0)
        pltpu.async_copy(
            tab_hbm.at[ridx.at[pl.ds(CH * 8, CH * 8)]], rows1, sr1)

        bufs = ((rows0, out0, sr0, so0), (rows1, out1, sr1, so1))
        zvec = jnp.zeros((_L,), jnp.int32)

        def chunk_body(sidx, carry):
            for u in range(2):
                rows, outb, sr, so = bufs[u]
                ch = sidx * 2 + u
                pltpu.make_async_copy(
                    tab_hbm.at[ridx.at[pl.ds(0, CH * 8)]], rows, sr).wait()

                @pl.when(sidx >= 1)
                def _wait_prev_store():
                    pltpu.make_async_copy(
                        outb, out_hbm.at[pl.ds(0, CH * 2)], so).wait()

                wbase = ch * (CH * 4)
                for pp in range(CH):
                    wq = zvec + (wbase + 4 * pp)
                    w0 = plsc.load_gather(wgt_v, [wq])
                    w1 = plsc.load_gather(wgt_v, [wq + 1])
                    w2 = plsc.load_gather(wgt_v, [wq + 2])
                    w3 = plsc.load_gather(wgt_v, [wq + 3])
                    for hh in range(2):
                        for rr in range(8):
                            sl = pl.ds(rr * _L, _L)
                            acc = rows[8 * pp + hh, sl] * w0
                            acc = acc + rows[8 * pp + 2 + hh, sl] * w1
                            acc = acc + rows[8 * pp + 4 + hh, sl] * w2
                            acc = acc + rows[8 * pp + 6 + hh, sl] * w3
                            outb[2 * pp + hh, sl] = acc

                @pl.when(ch + 2 < n_chunk)
                def _prefetch_next():
                    pltpu.async_copy(
                        tab_hbm.at[ridx.at[pl.ds((ch + 2) * (CH * 8), CH * 8)]],
                        rows, sr)

                rowbase = (b * N + p0 + ch * CH) * 2
                pltpu.async_copy(outb, out_hbm.at[pl.ds(rowbase, CH * 2)], so)
            return carry

        lax.fori_loop(0, n_chunk // 2, chunk_body, 0)
        pltpu.make_async_copy(out0, out_hbm.at[pl.ds(0, CH * 2)], so0).wait()
        pltpu.make_async_copy(out1, out_hbm.at[pl.ds(0, CH * 2)], so1).wait()

    return k(table2, keypoints, consts)


def kernel(keypoints, bev_features, batch_size, bev_stride):
    B, N, _ = keypoints.shape
    _, C, H, W = bev_features.shape
    stride_f = jnp.asarray(bev_stride, jnp.float32)
    scale_f = jnp.asarray(batch_size, jnp.float32) / B
    consts = jnp.stack([jnp.full((_L,), 1.0, jnp.float32) * stride_f,
                        jnp.full((_L,), 1.0, jnp.float32) * scale_f])
    table2 = (bev_features
              .transpose(0, 2, 3, 1)
              .reshape(B * H * W * 2, 128))
    out = _row_gather(table2, keypoints, consts, B, C, N, H, W)
    return out.reshape(B, N, C)
